# baseline (speedup 1.0000x reference)
import jax
import jax.numpy as jnp
from jax import lax
from jax.experimental import pallas as pl
from jax.experimental.pallas import tpu as pltpu
from jax.experimental.pallas import tpu_sc as plsc

_B, _M, _N = 4, 4096, 4096
_NNZ = 167772
_NC, _NS, _L = 2, 16, 16
_NW = _NC * _NS
_TPG = _NW // _B
_CH = 20992
_NNZ_PAD = _CH * _TPG
_NSPLIT = 2
_SUB = _CH // _NSPLIT
_SLICE = _M // _TPG
_UNROLL = 8

_W_PRIMAL, _W_DUAL, _W_STAT, _W_COMP = 0.1, 0.1, 0.6, 0.2


def _sc_kkt(x_hbm, lam_hbm, vals_hbm, rc_hbm, b_hbm, c_hbm,
            out_hbm,
            vals_v, rc_v, x_v, lam_v, ax_v, atl_v,
            bufa_v, bufb_v, b_v, c_v, outv,
            part_ax, part_atl, sh_x, sh_lam, sem_a, sem_b, sem_x):
    c = lax.axis_index("c")
    s = lax.axis_index("s")
    p = c * 2 + s // _TPG
    j = s % _TPG
    g0 = (s // _TPG) * _TPG
    wid = c * _NS + s
    pi = s // _TPG
    nz_base = p * _NNZ_PAD + j * _CH
    scope = jax.named_scope

    @pl.when(j == 0)
    def _stage_xl():
        pltpu.async_copy(x_hbm.at[pl.ds(p * _N, _N)], sh_x.at[pi], sem_x)
        pltpu.async_copy(lam_hbm.at[pl.ds(p * _M, _M)], sh_lam.at[pi], sem_x)

    cps_bc = [
        pltpu.async_copy(b_hbm.at[pl.ds(p * _M + j * _SLICE, _SLICE)], b_v, sem_x),
        pltpu.async_copy(c_hbm.at[pl.ds(p * _N + j * _SLICE, _SLICE)], c_v, sem_x),
    ]

    def fire(q, csem):
        base = nz_base + q * _SUB
        dst = pl.ds(q * _SUB, _SUB)
        return [
            pltpu.async_copy(vals_hbm.at[pl.ds(base, _SUB)],
                             vals_v.at[dst], csem),
            pltpu.async_copy(rc_hbm.at[pl.ds(base, _SUB)],
                             rc_v.at[dst], csem),
        ]

    cps_q0 = fire(0, sem_a)

    zero16 = jnp.zeros((_L,), jnp.float32)

    def zero_body(off):
        ax_v[pl.ds(off, _L)] = zero16
        atl_v[pl.ds(off, _L)] = zero16

    with jax.named_scope("p0_zero"):
        plsc.parallel_loop(0, _M, _L, unroll=8)(zero_body)

    with jax.named_scope("p0_bcast"):
        @pl.when(j == 0)
        def _wait_xl():
            pltpu.make_async_copy(x_hbm.at[pl.ds(p * _N, _N)],
                                  sh_x.at[pi], sem_x).wait()
            pltpu.make_async_copy(lam_hbm.at[pl.ds(p * _M, _M)],
                                  sh_lam.at[pi], sem_x).wait()
        plsc.subcore_barrier()
        cp1 = pltpu.async_copy(sh_x.at[pi], x_v, sem_x)
        cp2 = pltpu.async_copy(sh_lam.at[pi], lam_v, sem_x)
        cp1.wait()
        cp2.wait()

    with jax.named_scope("p0_wait"):
        for cp in cps_q0:
            cp.wait()

    cps_rest = []
    for q in range(1, _NSPLIT):
        cps_rest += fire(q, sem_b)

    def nnz_body(off):
        v16 = vals_v[pl.ds(off, _L)]
        rc16 = rc_v[pl.ds(off, _L)]
        r16 = rc16 & 0xFFFF
        k16 = lax.shift_right_logical(rc16, 16)
        xg = plsc.load_gather(x_v, [k16])
        plsc.addupdate_scatter(ax_v, [r16], v16 * xg)
        lg = plsc.load_gather(lam_v, [r16])
        plsc.addupdate_scatter(atl_v, [k16], v16 * lg)

    with jax.named_scope("p1_spmm"):
        plsc.parallel_loop(0, _SUB, _L, unroll=_UNROLL)(nnz_body)
        for cp in cps_rest:
            cp.wait()
        plsc.parallel_loop(_SUB, _CH, _L, unroll=_UNROLL)(nnz_body)

    with jax.named_scope("p2_pub"):
        cp1 = pltpu.async_copy(ax_v, part_ax.at[s], sem_x)
        cp2 = pltpu.async_copy(atl_v, part_atl.at[s], sem_x)
        cp1.wait()
        cp2.wait()
        plsc.subcore_barrier()

    off = j * _SLICE
    cps = [
        pltpu.async_copy(
            part_ax.at[pl.ds(g0, _TPG), pl.ds(off, _SLICE)], bufa_v, sem_x),
        pltpu.async_copy(
            part_atl.at[pl.ds(g0, _TPG), pl.ds(off, _SLICE)], bufb_v, sem_x),
    ]
    with jax.named_scope("p2_pull"):
        for cp in cps:
            cp.wait()

    def loss_body(t, acc):
        acc_p, acc_d, acc_s, acc_c = acc
        ds16 = pl.ds(t * _L, _L)
        ax16 = bufa_v[0, ds16]
        atl16 = bufb_v[0, ds16]
        for k in range(1, _TPG):
            ax16 = ax16 + bufa_v[k, ds16]
            atl16 = atl16 + bufb_v[k, ds16]
        b16 = b_v[ds16]
        c16 = c_v[ds16]
        lam16 = lam_v[pl.ds(off + t * _L, _L)]
        axmb = ax16 - b16
        relu_axmb = jnp.maximum(axmb, 0.0)
        relu_nlam = jnp.maximum(-lam16, 0.0)
        st = atl16 + c16
        cp16 = lam16 * axmb
        return (acc_p + relu_axmb * relu_axmb,
                acc_d + relu_nlam * relu_nlam,
                acc_s + st * st,
                acc_c + cp16 * cp16)

    acc0 = (zero16, zero16, zero16, zero16)
    for cp in cps_bc:
        cp.wait()
    with jax.named_scope("p3_loss"):
        acc_p, acc_d, acc_s, acc_c = lax.fori_loop(
            0, _SLICE // _L, loss_body, acc0)

    scale = 1.0 / (_M * _B)
    outv[...] = (_W_PRIMAL * acc_p + _W_DUAL * acc_d
                 + _W_STAT * acc_s + _W_COMP * acc_c) * scale
    pltpu.async_copy(outv, out_hbm.at[pl.ds(wid * _L, _L)], sem_x).wait()


@jax.jit
def _run(x_hat, lam_hat, vals_f, rc_f, b_f, c_f):
    mesh = plsc.VectorSubcoreMesh(core_axis_name="c", subcore_axis_name="s",
                                  num_cores=_NC, num_subcores=_NS)
    kern = pl.kernel(
        _sc_kkt,
        out_type=jax.ShapeDtypeStruct((_NW * _L,), jnp.float32),
        mesh=mesh,
        scratch_types=[
            pltpu.VMEM((_CH,), jnp.float32),
            pltpu.VMEM((_CH,), jnp.int32),
            pltpu.VMEM((_N,), jnp.float32),
            pltpu.VMEM((_M,), jnp.float32),
            pltpu.VMEM((_M,), jnp.float32),
            pltpu.VMEM((_N,), jnp.float32),
            pltpu.VMEM((_TPG, _SLICE), jnp.float32),
            pltpu.VMEM((_TPG, _SLICE), jnp.float32),
            pltpu.VMEM((_SLICE,), jnp.float32),
            pltpu.VMEM((_SLICE,), jnp.float32),
            pltpu.VMEM((_L,), jnp.float32),
            pltpu.VMEM_SHARED((_NS, _M), jnp.float32),
            pltpu.VMEM_SHARED((_NS, _N), jnp.float32),
            pltpu.VMEM_SHARED((2, _N), jnp.float32),
            pltpu.VMEM_SHARED((2, _M), jnp.float32),
            pltpu.SemaphoreType.DMA,
            pltpu.SemaphoreType.DMA,
            pltpu.SemaphoreType.DMA,
        ],
        compiler_params=pltpu.CompilerParams(needs_layout_passes=False),
    )
    out = kern(x_hat, lam_hat, vals_f, rc_f, b_f, c_f)
    return jnp.sum(out)


def kernel(x_hat, lam_hat, A_vals, A_rows, A_cols, b_pad, c_pad):
    pad = _NNZ_PAD - _NNZ
    vals_f = jnp.pad(A_vals, ((0, 0), (0, pad))).reshape(-1)
    rc = A_rows.astype(jnp.int32) | (A_cols.astype(jnp.int32) << 16)
    rc_f = jnp.pad(rc, ((0, 0), (0, pad))).reshape(-1)
    return _run(x_hat.astype(jnp.float32), lam_hat.astype(jnp.float32),
                vals_f, rc_f,
                b_pad.reshape(-1).astype(jnp.float32),
                c_pad.reshape(-1).astype(jnp.float32))


# speedup vs baseline: 1.0040x; 1.0040x over previous
import jax
import jax.numpy as jnp
from jax import lax
from jax.experimental import pallas as pl
from jax.experimental.pallas import tpu as pltpu
from jax.experimental.pallas import tpu_sc as plsc

_B, _M, _N = 4, 4096, 4096
_NNZ = 167772
_NC, _NS, _L = 2, 16, 16
_NW = _NC * _NS
_TPG = _NW // _B
_CH = 20992
_NNZ_PAD = _CH * _TPG
_NSPLIT = 4
_SUB = _CH // _NSPLIT
_SLICE = _M // _TPG
_UNROLL = 8

_W_PRIMAL, _W_DUAL, _W_STAT, _W_COMP = 0.1, 0.1, 0.6, 0.2


def _sc_kkt(x_hbm, lam_hbm, vals_hbm, rc_hbm, b_hbm, c_hbm,
            out_hbm,
            vals_v, rc_v, x_v, lam_v, ax_v, atl_v,
            bufa_v, bufb_v, b_v, c_v, outv,
            part_ax, part_atl, sh_x, sh_lam, sem_a, sem_b, sem_x):
    c = lax.axis_index("c")
    s = lax.axis_index("s")
    p = c * 2 + s // _TPG
    j = s % _TPG
    g0 = (s // _TPG) * _TPG
    wid = c * _NS + s
    pi = s // _TPG
    nz_base = p * _NNZ_PAD + j * _CH
    scope = jax.named_scope

    @pl.when(j == 0)
    def _stage_xl():
        pltpu.async_copy(x_hbm.at[pl.ds(p * _N, _N)], sh_x.at[pi], sem_x)
        pltpu.async_copy(lam_hbm.at[pl.ds(p * _M, _M)], sh_lam.at[pi], sem_x)

    cps_bc = [
        pltpu.async_copy(b_hbm.at[pl.ds(p * _M + j * _SLICE, _SLICE)], b_v, sem_x),
        pltpu.async_copy(c_hbm.at[pl.ds(p * _N + j * _SLICE, _SLICE)], c_v, sem_x),
    ]

    def fire(q, csem):
        base = nz_base + q * _SUB
        dst = pl.ds(q * _SUB, _SUB)
        return [
            pltpu.async_copy(vals_hbm.at[pl.ds(base, _SUB)],
                             vals_v.at[dst], csem),
            pltpu.async_copy(rc_hbm.at[pl.ds(base, _SUB)],
                             rc_v.at[dst], csem),
        ]

    cps_q0 = fire(0, sem_a)

    zero16 = jnp.zeros((_L,), jnp.float32)

    def zero_body(off):
        ax_v[pl.ds(off, _L)] = zero16
        atl_v[pl.ds(off, _L)] = zero16

    with jax.named_scope("p0_zero"):
        plsc.parallel_loop(0, _M, _L, unroll=8)(zero_body)

    with jax.named_scope("p0_bcast"):
        @pl.when(j == 0)
        def _wait_xl():
            pltpu.make_async_copy(x_hbm.at[pl.ds(p * _N, _N)],
                                  sh_x.at[pi], sem_x).wait()
            pltpu.make_async_copy(lam_hbm.at[pl.ds(p * _M, _M)],
                                  sh_lam.at[pi], sem_x).wait()
        plsc.subcore_barrier()
        cp1 = pltpu.async_copy(sh_x.at[pi], x_v, sem_x)
        cp2 = pltpu.async_copy(sh_lam.at[pi], lam_v, sem_x)
        cp1.wait()
        cp2.wait()

    with jax.named_scope("p0_wait"):
        for cp in cps_q0:
            cp.wait()

    cps_rest = []
    for q in range(1, _NSPLIT):
        cps_rest += fire(q, sem_b)

    def nnz_body(off):
        v16 = vals_v[pl.ds(off, _L)]
        rc16 = rc_v[pl.ds(off, _L)]
        r16 = rc16 & 0xFFFF
        k16 = lax.shift_right_logical(rc16, 16)
        xg = plsc.load_gather(x_v, [k16])
        plsc.addupdate_scatter(ax_v, [r16], v16 * xg)
        lg = plsc.load_gather(lam_v, [r16])
        plsc.addupdate_scatter(atl_v, [k16], v16 * lg)

    with jax.named_scope("p1_spmm"):
        plsc.parallel_loop(0, _SUB, _L, unroll=_UNROLL)(nnz_body)
        for cp in cps_rest:
            cp.wait()
        plsc.parallel_loop(_SUB, _CH, _L, unroll=_UNROLL)(nnz_body)

    with jax.named_scope("p2_pub"):
        cp1 = pltpu.async_copy(ax_v, part_ax.at[s], sem_x)
        cp2 = pltpu.async_copy(atl_v, part_atl.at[s], sem_x)
        cp1.wait()
        cp2.wait()
        plsc.subcore_barrier()

    off = j * _SLICE
    cps = [
        pltpu.async_copy(
            part_ax.at[pl.ds(g0, _TPG), pl.ds(off, _SLICE)], bufa_v, sem_x),
        pltpu.async_copy(
            part_atl.at[pl.ds(g0, _TPG), pl.ds(off, _SLICE)], bufb_v, sem_x),
    ]
    with jax.named_scope("p2_pull"):
        for cp in cps:
            cp.wait()

    def loss_body(t, acc):
        acc_p, acc_d, acc_s, acc_c = acc
        ds16 = pl.ds(t * _L, _L)
        ax16 = bufa_v[0, ds16]
        atl16 = bufb_v[0, ds16]
        for k in range(1, _TPG):
            ax16 = ax16 + bufa_v[k, ds16]
            atl16 = atl16 + bufb_v[k, ds16]
        b16 = b_v[ds16]
        c16 = c_v[ds16]
        lam16 = lam_v[pl.ds(off + t * _L, _L)]
        axmb = ax16 - b16
        relu_axmb = jnp.maximum(axmb, 0.0)
        relu_nlam = jnp.maximum(-lam16, 0.0)
        st = atl16 + c16
        cp16 = lam16 * axmb
        return (acc_p + relu_axmb * relu_axmb,
                acc_d + relu_nlam * relu_nlam,
                acc_s + st * st,
                acc_c + cp16 * cp16)

    acc0 = (zero16, zero16, zero16, zero16)
    for cp in cps_bc:
        cp.wait()
    with jax.named_scope("p3_loss"):
        acc_p, acc_d, acc_s, acc_c = lax.fori_loop(
            0, _SLICE // _L, loss_body, acc0)

    scale = 1.0 / (_M * _B)
    outv[...] = (_W_PRIMAL * acc_p + _W_DUAL * acc_d
                 + _W_STAT * acc_s + _W_COMP * acc_c) * scale
    pltpu.async_copy(outv, out_hbm.at[pl.ds(wid * _L, _L)], sem_x).wait()


@jax.jit
def _run(x_hat, lam_hat, vals_f, rc_f, b_f, c_f):
    mesh = plsc.VectorSubcoreMesh(core_axis_name="c", subcore_axis_name="s",
                                  num_cores=_NC, num_subcores=_NS)
    kern = pl.kernel(
        _sc_kkt,
        out_type=jax.ShapeDtypeStruct((_NW * _L,), jnp.float32),
        mesh=mesh,
        scratch_types=[
            pltpu.VMEM((_CH,), jnp.float32),
            pltpu.VMEM((_CH,), jnp.int32),
            pltpu.VMEM((_N,), jnp.float32),
            pltpu.VMEM((_M,), jnp.float32),
            pltpu.VMEM((_M,), jnp.float32),
            pltpu.VMEM((_N,), jnp.float32),
            pltpu.VMEM((_TPG, _SLICE), jnp.float32),
            pltpu.VMEM((_TPG, _SLICE), jnp.float32),
            pltpu.VMEM((_SLICE,), jnp.float32),
            pltpu.VMEM((_SLICE,), jnp.float32),
            pltpu.VMEM((_L,), jnp.float32),
            pltpu.VMEM_SHARED((_NS, _M), jnp.float32),
            pltpu.VMEM_SHARED((_NS, _N), jnp.float32),
            pltpu.VMEM_SHARED((2, _N), jnp.float32),
            pltpu.VMEM_SHARED((2, _M), jnp.float32),
            pltpu.SemaphoreType.DMA,
            pltpu.SemaphoreType.DMA,
            pltpu.SemaphoreType.DMA,
        ],
        compiler_params=pltpu.CompilerParams(needs_layout_passes=False),
    )
    out = kern(x_hat, lam_hat, vals_f, rc_f, b_f, c_f)
    return jnp.sum(out)


def kernel(x_hat, lam_hat, A_vals, A_rows, A_cols, b_pad, c_pad):
    pad = _NNZ_PAD - _NNZ
    vals_f = jnp.pad(A_vals, ((0, 0), (0, pad))).reshape(-1)
    rc = A_rows.astype(jnp.int32) | (A_cols.astype(jnp.int32) << 16)
    rc_f = jnp.pad(rc, ((0, 0), (0, pad))).reshape(-1)
    return _run(x_hat.astype(jnp.float32), lam_hat.astype(jnp.float32),
                vals_f, rc_f,
                b_pad.reshape(-1).astype(jnp.float32),
                c_pad.reshape(-1).astype(jnp.float32))
